# baseline (speedup 1.0000x reference)
import jax
import jax.numpy as jnp
from jax.experimental import pallas as pl

N = 10000
H = 64


def _elu_pallas(x):
    def body(x_ref, o_ref):
        v = x_ref[...]
        o_ref[...] = jnp.where(v > 0, v, jnp.exp(jnp.minimum(v, 0.0)) - 1.0)

    return pl.pallas_call(
        body,
        out_shape=jax.ShapeDtypeStruct(x.shape, x.dtype),
    )(x)


def _gat_attn(x, W, a_src, a_dst, src, dst, n):
    h = x @ W
    alpha_src = h @ a_src
    alpha_dst = h @ a_dst
    e = alpha_src[src] + alpha_dst[dst]
    e = jnp.where(e > 0, e, 0.2 * e)
    m = jax.ops.segment_max(e, dst, num_segments=n)
    m = jnp.where(jnp.isfinite(m), m, 0.0)
    ex = jnp.exp(e - m[dst])
    ssum = jax.ops.segment_sum(ex, dst, num_segments=n)
    attn = ex / (ssum[dst] + 1e-16)
    out = jax.ops.segment_sum(h[src] * attn[:, None], dst, num_segments=n)
    return out


def kernel(features, im_features, edge_index, W1x, a1x_src, a1x_dst, W1r, a1r_src, a1r_dst, W2, a3_src, a3_dst):
    n = features.shape[0]
    src = edge_index[0]
    dst = edge_index[1]
    perm = jax.random.permutation(jax.random.key(42), n)
    rand_features = features[perm]
    rand_im_features = im_features[perm]

    def encode(fx, fr):
        h1x = _elu_pallas(_gat_attn(fx, W1x, a1x_src, a1x_dst, src, dst, n))
        h1r = _elu_pallas(_gat_attn(fr, W1r, a1r_src, a1r_dst, src, dst, n))
        t = jnp.concatenate([h1x, h1r], axis=1)
        h2 = t @ W2
        h3 = _elu_pallas(_gat_attn(h2, W2.T, a3_src, a3_dst, src, dst, n))
        k1 = h3[:, :H]
        k2 = h3[:, H:]
        h4x = k1 @ W1x.T
        h4r = k2 @ W1r.T
        return h2, h4x, h4r

    h2, h4x, h4r = encode(features, im_features)
    rand_h2, rand_h4x, rand_h4r = encode(rand_features, rand_im_features)
    summary = jax.nn.sigmoid(h2.mean(axis=0))
    return (h2, h4x, h4r, rand_h2, rand_h4x, rand_h4r, summary)


# speedup vs baseline: 11.3000x; 11.3000x over previous
import functools

import jax
import jax.numpy as jnp
from jax import lax
from jax.experimental import pallas as pl
from jax.experimental.pallas import tpu as pltpu
from jax.experimental.pallas import tpu_sc as plsc

N = 10000
E = 320000
H = 64
NT = 32
ET = 10240
EP = NT * ET
C = 128
NCH = ET // C
NA = 10240
NPAD = 10112
RW = 128
SW = 16
DR = NPAD // 16

_f32 = jnp.float32


_mesh = plsc.VectorSubcoreMesh(
    core_axis_name="c", subcore_axis_name="s", num_cores=2, num_subcores=16
)


@functools.partial(
    pl.kernel,
    out_type=[jax.ShapeDtypeStruct((NT, NCH, C), _f32),
              jax.ShapeDtypeStruct((NT, NCH, C), _f32)],
    mesh=_mesh,
    scratch_types=[
        pltpu.VMEM((NA,), _f32),
        pltpu.VMEM((NA,), _f32),
        pltpu.VMEM((NA,), _f32),
        pltpu.VMEM((NA,), _f32),
        pltpu.VMEM((NCH, C), jnp.int32),
        pltpu.VMEM((NCH, C), jnp.int32),
        pltpu.VMEM((NCH, C), _f32),
        pltpu.VMEM((NCH, C), _f32),
        pltpu.VMEM((16,), _f32),
    ],
    compiler_params=pltpu.CompilerParams(needs_layout_passes=False),
)
def _sc_alpha(src_hbm, dst_hbm, alpha_hbm, g_hbm, exx_hbm, exr_hbm,
              asx_v, adx_v, asr_v, adr_v, src_v, dst_v,
              exx_v, exr_v, g_v):
    c = lax.axis_index("c")
    s = lax.axis_index("s")
    wid = s * 2 + c

    pltpu.sync_copy(src_hbm.at[wid], src_v)
    pltpu.sync_copy(dst_hbm.at[wid], dst_v)
    pltpu.sync_copy(alpha_hbm.at[0], asx_v)
    pltpu.sync_copy(alpha_hbm.at[1], adx_v)
    pltpu.sync_copy(alpha_hbm.at[2], asr_v)
    pltpu.sync_copy(alpha_hbm.at[3], adr_v)
    pltpu.sync_copy(g_hbm, g_v)
    gvec = g_v[...]
    gx = gvec[0]
    gr = gvec[1]

    def _phA(j, carry):
        for kk in range(C // 16):
            s16 = src_v[j, pl.ds(kk * 16, 16)]
            d16 = dst_v[j, pl.ds(kk * 16, 16)]
            asx = plsc.load_gather(asx_v, [s16])
            adx = plsc.load_gather(adx_v, [d16])
            asr = plsc.load_gather(asr_v, [s16])
            adr = plsc.load_gather(adr_v, [d16])
            e1 = asx + adx
            e1 = jnp.where(e1 > 0, e1, 0.2 * e1)
            e2 = asr + adr
            e2 = jnp.where(e2 > 0, e2, 0.2 * e2)
            exx_v[j, pl.ds(kk * 16, 16)] = jnp.exp(e1 - gx)
            exr_v[j, pl.ds(kk * 16, 16)] = jnp.exp(e2 - gr)
        return carry

    lax.fori_loop(0, NCH, _phA, 0)

    pltpu.sync_copy(exx_v, exx_hbm.at[wid])
    pltpu.sync_copy(exr_v, exr_hbm.at[wid])


@functools.partial(
    pl.kernel,
    out_type=jax.ShapeDtypeStruct((2, NPAD, RW), _f32),
    mesh=_mesh,
    scratch_types=[
        pltpu.VMEM((C,), jnp.int32),
        pltpu.VMEM((C,), jnp.int32),
        pltpu.VMEM((C,), _f32),
        pltpu.VMEM((C,), _f32),
        pltpu.VMEM((C, 128), _f32),
        pltpu.VMEM((C, RW), _f32),
        pltpu.VMEM_SHARED((NPAD, RW), _f32),
    ],
    compiler_params=pltpu.CompilerParams(needs_layout_passes=False),
)
def _sc_rows(src_hbm, dst_hbm, exx_hbm, exr_hbm, h_hbm, out_hbm,
             srcc_v, dstc_v, exxc_v, exrc_v, rows_v, scat_v, acc_sh):
    c = lax.axis_index("c")
    s = lax.axis_index("s")
    wid = s * 2 + c

    zero16 = jnp.zeros((16,), _f32)

    def _zrow(i, carry):
        for k in range(RW // 16):
            scat_v[i, pl.ds(k * 16, 16)] = zero16
        return carry

    lax.fori_loop(0, C, _zrow, 0)
    for t in range(DR // C):
        pltpu.sync_copy(scat_v, acc_sh.at[pl.ds(s * DR + t * C, C)])
    rem = DR % C
    if rem:
        pltpu.sync_copy(scat_v.at[pl.ds(0, rem)],
                        acc_sh.at[pl.ds(s * DR + (DR // C) * C, rem)])
    plsc.subcore_barrier()

    def _phB(j, carry):
        pltpu.sync_copy(src_hbm.at[wid, j], srcc_v)
        pltpu.sync_copy(dst_hbm.at[wid, j], dstc_v)
        pltpu.sync_copy(exx_hbm.at[wid, j], exxc_v)
        pltpu.sync_copy(exr_hbm.at[wid, j], exrc_v)
        pltpu.sync_copy(h_hbm.at[srcc_v], rows_v)

        def _grp(k, icarry):
            ex16x = exxc_v[pl.ds(k * 16, 16)]
            ex16r = exrc_v[pl.ds(k * 16, 16)]
            for t in range(16):
                i = k * 16 + t
                bx = jnp.full((16,), ex16x[t], _f32)
                br = jnp.full((16,), ex16r[t], _f32)
                for kc in range(4):
                    scat_v[i, pl.ds(kc * 16, 16)] = rows_v[i, pl.ds(kc * 16, 16)] * bx
                for kc in range(4, 8):
                    scat_v[i, pl.ds(kc * 16, 16)] = rows_v[i, pl.ds(kc * 16, 16)] * br
            return icarry

        lax.fori_loop(0, C // 16, _grp, 0)
        pltpu.sync_copy(scat_v, acc_sh.at[dstc_v], add=True)
        return carry

    lax.fori_loop(0, NCH, _phB, 0)
    plsc.subcore_barrier()

    pltpu.sync_copy(acc_sh.at[pl.ds(s * DR, DR)], out_hbm.at[c, pl.ds(s * DR, DR)])


_B = 1000
_G = N // _B


def _elu(v):
    return jnp.where(v > 0, v, jnp.exp(jnp.minimum(v, 0.0)) - 1.0)


def _tc1_body(fx_ref, fr_ref, w1x_ref, w1r_ref, ast_ref,
              hcat_ref, al_ref, mx_ref):
    hx = fx_ref[...] @ w1x_ref[...]
    hr = fr_ref[...] @ w1r_ref[...]
    hcat = jnp.concatenate([hx, hr], axis=1)
    hcat_ref[...] = hcat
    al = hcat @ ast_ref[...]
    al_ref[...] = al
    mx_ref[...] = jnp.broadcast_to(jnp.max(al, axis=0)[None, None, :], (1, 8, 8))


def _tc1(fx, fr, w1x, w1r, ast):
    return pl.pallas_call(
        _tc1_body,
        grid=(_G,),
        in_specs=[
            pl.BlockSpec((_B, 128), lambda i: (i, 0)),
            pl.BlockSpec((_B, 128), lambda i: (i, 0)),
            pl.BlockSpec((128, H), lambda i: (0, 0)),
            pl.BlockSpec((128, H), lambda i: (0, 0)),
            pl.BlockSpec((128, 8), lambda i: (0, 0)),
        ],
        out_specs=[
            pl.BlockSpec((_B, 128), lambda i: (i, 0)),
            pl.BlockSpec((_B, 8), lambda i: (i, 0)),
            pl.BlockSpec((1, 8, 8), lambda i: (i, 0, 0)),
        ],
        out_shape=[
            jax.ShapeDtypeStruct((N, 128), _f32),
            jax.ShapeDtypeStruct((N, 8), _f32),
            jax.ShapeDtypeStruct((_G, 8, 8), _f32),
        ],
    )(fx, fr, w1x, w1r, ast)


def _norm_halves(a, ss):
    sx = ss[:, 0:1] + 1e-30
    sr = ss[:, 1:2] + 1e-30
    return _elu(a[:, :64] / sx), _elu(a[:, 64:128] / sr)


def _tc2_body(acc_ref, accs_ref, w2_ref, w2t_ref, a3_ref,
              h2_ref, h3p_ref, al3_ref, mx_ref, cs_ref):
    a = acc_ref[0] + acc_ref[1]
    ss = accs_ref[0] + accs_ref[1]
    h1x, h1r = _norm_halves(a, ss)
    t = jnp.concatenate([h1x, h1r], axis=1)
    h2 = t @ w2_ref[...]
    h2_ref[...] = h2
    h3p = h2 @ w2t_ref[...]
    h3p_ref[...] = h3p
    al3 = h3p @ a3_ref[...]
    al3_ref[...] = al3
    mx_ref[...] = jnp.broadcast_to(jnp.max(al3, axis=0)[None, None, :], (1, 8, 8))
    cs_ref[...] = jnp.broadcast_to(jnp.sum(h2, axis=0)[None, None, :], (1, 8, 32))


def _tc2(acc, accs, w2, w2t, a3):
    return pl.pallas_call(
        _tc2_body,
        grid=(_G,),
        in_specs=[
            pl.BlockSpec((2, _B, RW), lambda i: (0, i, 0)),
            pl.BlockSpec((2, _B, SW), lambda i: (0, i, 0)),
            pl.BlockSpec((128, 32), lambda i: (0, 0)),
            pl.BlockSpec((32, 128), lambda i: (0, 0)),
            pl.BlockSpec((128, 8), lambda i: (0, 0)),
        ],
        out_specs=[
            pl.BlockSpec((_B, 32), lambda i: (i, 0)),
            pl.BlockSpec((_B, 128), lambda i: (i, 0)),
            pl.BlockSpec((_B, 8), lambda i: (i, 0)),
            pl.BlockSpec((1, 8, 8), lambda i: (i, 0, 0)),
            pl.BlockSpec((1, 8, 32), lambda i: (i, 0, 0)),
        ],
        out_shape=[
            jax.ShapeDtypeStruct((N, 32), _f32),
            jax.ShapeDtypeStruct((N, 128), _f32),
            jax.ShapeDtypeStruct((N, 8), _f32),
            jax.ShapeDtypeStruct((_G, 8, 8), _f32),
            jax.ShapeDtypeStruct((_G, 8, 32), _f32),
        ],
    )(acc, accs, w2, w2t, a3)


def _tc3_body(acc_ref, accs_ref, w1xt_ref, w1rt_ref, h4x_ref, h4r_ref):
    a = acc_ref[0] + acc_ref[1]
    ss = accs_ref[0] + accs_ref[1]
    h3x, h3r = _norm_halves(a, ss)
    h4x_ref[...] = h3x @ w1xt_ref[...]
    h4r_ref[...] = h3r @ w1rt_ref[...]


def _tc3(acc, accs, w1xt, w1rt):
    return pl.pallas_call(
        _tc3_body,
        grid=(_G,),
        in_specs=[
            pl.BlockSpec((2, _B, RW), lambda i: (0, i, 0)),
            pl.BlockSpec((2, _B, SW), lambda i: (0, i, 0)),
            pl.BlockSpec((H, 128), lambda i: (0, 0)),
            pl.BlockSpec((H, 128), lambda i: (0, 0)),
        ],
        out_specs=[
            pl.BlockSpec((_B, 128), lambda i: (i, 0)),
            pl.BlockSpec((_B, 128), lambda i: (i, 0)),
        ],
        out_shape=[
            jax.ShapeDtypeStruct((N, 128), _f32),
            jax.ShapeDtypeStruct((N, 128), _f32),
        ],
    )(acc, accs, w1xt, w1rt)


def _alpha_table(al):
    return jnp.zeros((4, NA), _f32).at[:, :N].set(al[:, :4].T)


def _gvec(mx):
    gx = jnp.maximum(jnp.max(mx[:, 0]) + jnp.max(mx[:, 1]), 0.0)
    gr = jnp.maximum(jnp.max(mx[:, 2]) + jnp.max(mx[:, 3]), 0.0)
    return jnp.zeros((16,), _f32).at[0].set(gx).at[1].set(gr)


def kernel(features, im_features, edge_index, W1x, a1x_src, a1x_dst,
           W1r, a1r_src, a1r_dst, W2, a3_src, a3_dst):
    src = edge_index[0]
    dst = edge_index[1]
    srcp = jnp.concatenate(
        [src, jnp.zeros((EP - E,), jnp.int32)]).reshape(NT, NCH, C)
    dstp = jnp.concatenate(
        [dst, jnp.full((EP - E,), N, jnp.int32)]).reshape(NT, NCH, C)

    z64 = jnp.zeros((H,), _f32)
    z128 = jnp.zeros((128,), _f32)
    ast1 = jnp.stack([
        jnp.concatenate([a1x_src, z64]), jnp.concatenate([a1x_dst, z64]),
        jnp.concatenate([z64, a1r_src]), jnp.concatenate([z64, a1r_dst]),
        z128, z128, z128, z128], axis=1)
    a3t = jnp.stack([a3_src, a3_dst, a3_src, a3_dst,
                     z128, z128, z128, z128], axis=1)
    w2t = W2.T
    w1xt = W1x.T
    w1rt = W1r.T

    perm = jax.random.permutation(jax.random.key(42), N)

    def gat_pass(h, al, mx):
        exx, exr = _sc_alpha(srcp, dstp, _alpha_table(al), _gvec(mx))
        acc = _sc_rows(srcp, dstp, exx, exr, h)
        exf = exx.reshape(EP)[:E]
        exfr = exr.reshape(EP)[:E]
        ssx = jax.ops.segment_sum(exf, dst, num_segments=N)
        ssr = jax.ops.segment_sum(exfr, dst, num_segments=N)
        accsp = jnp.zeros((2, N, SW), _f32).at[0, :, 0].set(ssx).at[0, :, 1].set(ssr)
        return acc[:, :N, :], accsp

    def encode(fx, fr):
        hcat, al, mx = _tc1(fx, fr, W1x, W1r, ast1)
        acc1, accs1 = gat_pass(hcat, al, mx[:, 0, :])
        h2, h3p, al3, mx3, cs = _tc2(acc1, accs1, W2, w2t, a3t)
        acc3, accs3 = gat_pass(h3p, al3, mx3[:, 0, :])
        h4x, h4r = _tc3(acc3, accs3, w1xt, w1rt)
        return h2, h4x, h4r, cs[:, 0, :]

    h2, h4x, h4r, cs = encode(features, im_features)
    rh2, rh4x, rh4r, _ = encode(features[perm], im_features[perm])
    summary = jax.nn.sigmoid(jnp.sum(cs, axis=0) / N)
    return (h2, h4x, h4r, rh2, rh4x, rh4r, summary)
